# baseline (speedup 1.0000x reference)
import jax
import jax.numpy as jnp
from jax import lax
from jax.experimental import pallas as pl
from jax.experimental.pallas import tpu as pltpu
from jax.experimental.pallas import tpu_sc as plsc

NC = 2
NS = 16
L = 16
NW = NC * NS

BATCH = 16384
DIM = 32
BPW = BATCH // NW
CHUNK = 128
NCHUNK = BPW // CHUNK
GROUPS = BPW // L


def _rsqrt16(x):
    i = plsc.bitcast(x, jnp.int32)
    i = jnp.int32(0x5F3759DF) - lax.shift_right_logical(i, 1)
    y = plsc.bitcast(i, jnp.float32)
    half_x = x * 0.5
    for _ in range(3):
        y = y * (1.5 - half_x * y * y)
    return y


def _sc_kernel(ni_hbm, nj_hbm, table_hbm, beta_hbm, out_hbm,
               idx_i, idx_j, rows_i, rows_j, out_v, beta_v, sem):
    wid = lax.axis_index("s") * NC + lax.axis_index("c")

    pltpu.sync_copy(ni_hbm.at[wid], idx_i)
    pltpu.sync_copy(nj_hbm.at[wid], idx_j)
    pltpu.sync_copy(beta_hbm, beta_v)

    copies = []
    for k in range(NCHUNK):
        copies.append(pltpu.async_copy(
            table_hbm.at[idx_i.at[k]], rows_i.at[pl.ds(k * CHUNK, CHUNK)], sem))
        copies.append(pltpu.async_copy(
            table_hbm.at[idx_j.at[k]], rows_j.at[pl.ds(k * CHUNK, CHUNK)], sem))
    for c in copies:
        c.wait()

    beta_vec = beta_v[...]
    lane = lax.iota(jnp.int32, L)

    def group_body(g, carry):
        row = g * L + lane
        acc = jnp.zeros((L,), jnp.float32)
        for d in range(DIM):
            col = jnp.full((L,), d, jnp.int32)
            vi = plsc.load_gather(rows_i, [row, col])
            vj = plsc.load_gather(rows_j, [row, col])
            diff = vi - vj
            acc = acc + diff * diff
        norm = acc * _rsqrt16(acc)
        out_v[pl.ds(g * L, L)] = beta_vec - norm
        return carry

    lax.fori_loop(0, GROUPS, group_body, 0)

    pltpu.sync_copy(out_v, out_hbm.at[pl.ds(wid * BPW, BPW)])


def kernel(node_i, node_j, table, beta):
    ni = node_i.astype(jnp.int32).reshape(NW, NCHUNK, CHUNK)
    nj = node_j.astype(jnp.int32).reshape(NW, NCHUNK, CHUNK)
    beta_vec = jnp.full((L,), beta, jnp.float32)

    mesh = plsc.VectorSubcoreMesh(core_axis_name="c", subcore_axis_name="s")
    run = pl.kernel(
        _sc_kernel,
        mesh=mesh,
        compiler_params=pltpu.CompilerParams(
            needs_layout_passes=False, use_tc_tiling_on_sc=False),
        out_type=jax.ShapeDtypeStruct((BATCH,), jnp.float32),
        scratch_types=[
            pltpu.VMEM((NCHUNK, CHUNK), jnp.int32),
            pltpu.VMEM((NCHUNK, CHUNK), jnp.int32),
            pltpu.VMEM((BPW, DIM), jnp.float32),
            pltpu.VMEM((BPW, DIM), jnp.float32),
            pltpu.VMEM((BPW,), jnp.float32),
            pltpu.VMEM((L,), jnp.float32),
            pltpu.SemaphoreType.DMA,
        ],
    )
    return run(ni, nj, table, beta_vec)


# speedup vs baseline: 2.4325x; 2.4325x over previous
import jax
import jax.numpy as jnp
from jax import lax
from jax.experimental import pallas as pl
from jax.experimental.pallas import tpu as pltpu
from jax.experimental.pallas import tpu_sc as plsc

NC = 2
NS = 16
L = 16
NW = NC * NS

BATCH = 16384
DIM = 32
ROWS = 1_000_000
NENT = 2 * BATCH

C = 1024
CSHIFT = 10
NFULL = ROWS // C
TAIL0 = NFULL * C
TAIL1 = TAIL0 + 512
FULL_ROUNDS = NFULL // NW

WL_CAP = 2048
RL_CAP = 1024
ZPAD = 128
ZGROUPS = ZPAD // L


def _rsqrt16(x):
    i = plsc.bitcast(x, jnp.int32)
    i = jnp.int32(0x5F3759DF) - lax.shift_right_logical(i, 1)
    y = plsc.bitcast(i, jnp.float32)
    half_x = x * 0.5
    for _ in range(3):
        y = y * (1.5 - half_x * y * y)
    return y


def _scalar(v16):
    return v16[0]


def _chunk_of(rr):
    c = lax.shift_right_logical(rr, CSHIFT)
    return c + jnp.where(rr >= TAIL1, jnp.int32(1), jnp.int32(0))


def _route_kernel(ni_hbm, nj_hbm, tab_t_hbm, tab_tail_hbm, zbuf_hbm,
                  nodes_v, chunk_v, wl_r, wl_d, rl_r, rl_d,
                  zst0, zst1, zix0, zix1, sem):
    wid = lax.axis_index("s") * NC + lax.axis_index("c")
    lane = lax.iota(jnp.int32, L)

    pltpu.sync_copy(ni_hbm, nodes_v.at[pl.ds(0, BATCH)])
    pltpu.sync_copy(nj_hbm, nodes_v.at[pl.ds(BATCH, BATCH)])

    def wl_body(v, off):
        rr = nodes_v[pl.ds(v * L, L)]
        m = (_chunk_of(rr) & (NW - 1)) == wid
        cnt = _scalar(plsc.all_reduce_population_count(m))
        plsc.store_compressed(wl_r.at[pl.ds(off, L)], rr, mask=m)
        plsc.store_compressed(wl_d.at[pl.ds(off, L)], v * L + lane, mask=m)
        return off + cnt

    n_w = lax.fori_loop(0, NENT // L, wl_body, jnp.int32(0))
    nv_w = (n_w + (L - 1)) // L

    def do_round(c, base, size, src):
        pltpu.sync_copy(src, chunk_v.at[:, pl.ds(0, size)])

        def rl_body(v, off):
            e = v * L + lane
            rr = wl_r[pl.ds(v * L, L)]
            dd = wl_d[pl.ds(v * L, L)]
            m = (_chunk_of(rr) == c) & (e < n_w)
            cnt = _scalar(plsc.all_reduce_population_count(m))
            plsc.store_compressed(rl_r.at[pl.ds(off, L)], rr - base, mask=m)
            plsc.store_compressed(rl_d.at[pl.ds(off, L)], dd, mask=m)
            return off + cnt

        n_r = lax.fori_loop(0, nv_w, rl_body, jnp.int32(0))

        def fill(e0, zst, zix):
            ng = lax.max(jnp.int32(0),
                         lax.min(jnp.int32(ZGROUPS),
                                 (n_r - e0 + (L - 1)) // L))

            def clear_body(g, carry):
                zix[pl.ds(g * L, L)] = jnp.full((L,), -1, jnp.int32)
                return carry

            lax.fori_loop(ng, ZGROUPS, clear_body, 0)

            def fill_body(g, carry):
                e = e0 + g * L
                lr = wl_clamp = rl_r[pl.ds(e, L)]
                lr = lax.max(jnp.int32(0),
                             lax.min(lr, jnp.int32(size - 1)))
                ld = rl_d[pl.ds(e, L)]
                ld = jnp.where(e + lane < n_r, ld, jnp.int32(-1))
                zix[pl.ds(g * L, L)] = ld
                row = g * L + lane
                for d in range(DIM):
                    col = jnp.full((L,), d, jnp.int32)
                    v = plsc.load_gather(chunk_v, [col, lr])
                    plsc.store_scatter(zst, [row, col], v)
                return carry

            lax.fori_loop(0, ng, fill_body, 0)

        def burst_pair(p, carry):
            e0 = p * (2 * ZPAD)
            fill(e0, zst0, zix0)
            c0 = pltpu.async_copy(
                zst0, zbuf_hbm.at[plsc.Indices(zix0, ignored_value=-1)], sem)

            @pl.when(e0 + ZPAD < n_r)
            def _():
                fill(e0 + ZPAD, zst1, zix1)
                pltpu.async_copy(
                    zst1, zbuf_hbm.at[plsc.Indices(zix1, ignored_value=-1)],
                    sem).wait()

            c0.wait()
            return carry

        nb = (n_r + (2 * ZPAD - 1)) // (2 * ZPAD)
        lax.fori_loop(0, nb, burst_pair, jnp.int32(0))

    def round_body(t, carry):
        c = wid + t * NW
        base = c * C
        do_round(c, base, C, tab_t_hbm.at[:, pl.ds(base, C)])
        return carry

    lax.fori_loop(0, FULL_ROUNDS, round_body, jnp.int32(0))

    last_c = wid + FULL_ROUNDS * NW

    @pl.when(wid < NFULL - FULL_ROUNDS * NW)
    def _():
        do_round(last_c, last_c * C, C,
                 tab_t_hbm.at[:, pl.ds(last_c * C, C)])

    @pl.when(wid == NFULL - FULL_ROUNDS * NW)
    def _():
        do_round(last_c, jnp.int32(TAIL0), 512,
                 tab_t_hbm.at[:, pl.ds(TAIL0, 512)])

    @pl.when(wid == NFULL - FULL_ROUNDS * NW + 1)
    def _():
        do_round(last_c, jnp.int32(TAIL1), 128, tab_tail_hbm)


def _norm_kernel(zbuf_hbm, beta_hbm, out_hbm, zi_v, zj_v, out_v, beta_v):
    wid = lax.axis_index("s") * NC + lax.axis_index("c")
    lane = lax.iota(jnp.int32, L)
    pltpu.sync_copy(beta_hbm, beta_v)
    beta_vec = beta_v[...]
    bpw = BATCH // NW
    slab = 128

    def slab_body(s, carry):
        b0 = wid * bpw + s * slab
        pltpu.sync_copy(zbuf_hbm.at[pl.ds(b0, slab)], zi_v)
        pltpu.sync_copy(zbuf_hbm.at[pl.ds(BATCH + b0, slab)], zj_v)

        def group_body(g, carry2):
            row = g * L + lane
            acc = jnp.zeros((L,), jnp.float32)
            for d in range(DIM):
                col = jnp.full((L,), d, jnp.int32)
                vi = plsc.load_gather(zi_v, [row, col])
                vj = plsc.load_gather(zj_v, [row, col])
                diff = vi - vj
                acc = acc + diff * diff
            norm = acc * _rsqrt16(acc)
            out_v[pl.ds(s * slab + g * L, L)] = beta_vec - norm
            return carry2

        lax.fori_loop(0, slab // L, group_body, 0)
        return carry

    lax.fori_loop(0, bpw // slab, slab_body, 0)
    pltpu.sync_copy(out_v, out_hbm.at[pl.ds(wid * bpw, bpw)])


def kernel(node_i, node_j, table, beta):
    ni = node_i.astype(jnp.int32)
    nj = node_j.astype(jnp.int32)
    beta_vec = jnp.full((L,), beta, jnp.float32)
    tab_t = table.T
    tab_tail = lax.pad(lax.slice(tab_t, (0, TAIL1), (DIM, ROWS)),
                       jnp.float32(0), ((0, 0, 0), (0, 64, 0)))

    mesh = plsc.VectorSubcoreMesh(core_axis_name="c", subcore_axis_name="s")
    params = pltpu.CompilerParams(
        needs_layout_passes=False, use_tc_tiling_on_sc=True)

    route = pl.kernel(
        _route_kernel,
        mesh=mesh,
        compiler_params=params,
        out_type=jax.ShapeDtypeStruct((NENT, ZPAD), jnp.float32),
        scratch_types=[
            pltpu.VMEM((NENT,), jnp.int32),
            pltpu.VMEM((DIM, C), jnp.float32),
            pltpu.VMEM((WL_CAP,), jnp.int32),
            pltpu.VMEM((WL_CAP,), jnp.int32),
            pltpu.VMEM((RL_CAP,), jnp.int32),
            pltpu.VMEM((RL_CAP,), jnp.int32),
            pltpu.VMEM((ZPAD, ZPAD), jnp.float32),
            pltpu.VMEM((ZPAD, ZPAD), jnp.float32),
            pltpu.VMEM((ZPAD,), jnp.int32),
            pltpu.VMEM((ZPAD,), jnp.int32),
            pltpu.SemaphoreType.DMA,
        ],
    )
    zbuf = route(ni, nj, tab_t, tab_tail)

    norm = pl.kernel(
        _norm_kernel,
        mesh=mesh,
        compiler_params=params,
        out_type=jax.ShapeDtypeStruct((BATCH,), jnp.float32),
        scratch_types=[
            pltpu.VMEM((128, ZPAD), jnp.float32),
            pltpu.VMEM((128, ZPAD), jnp.float32),
            pltpu.VMEM((BATCH // NW,), jnp.float32),
            pltpu.VMEM((L,), jnp.float32),
        ],
    )
    return norm(zbuf, beta_vec)


# speedup vs baseline: 3.2929x; 1.3537x over previous
import jax
import jax.numpy as jnp
from jax import lax
from jax.experimental import pallas as pl
from jax.experimental.pallas import tpu as pltpu
from jax.experimental.pallas import tpu_sc as plsc

NC = 2
NS = 16
L = 16
NW = NC * NS

BATCH = 16384
DIM = 32
ROWS = 1_000_000
NENT = 2 * BATCH

C = 1024
CSHIFT = 10
NFULL = ROWS // C
TAIL0 = NFULL * C
TAIL1 = TAIL0 + 512
FULL_ROUNDS = NFULL // NW

WL_CAP = 2048
RL_CAP = 1024
ZPAD = 128
ZGROUPS = ZPAD // L


def _rsqrt16(x):
    i = plsc.bitcast(x, jnp.int32)
    i = jnp.int32(0x5F3759DF) - lax.shift_right_logical(i, 1)
    y = plsc.bitcast(i, jnp.float32)
    half_x = x * 0.5
    for _ in range(3):
        y = y * (1.5 - half_x * y * y)
    return y


def _scalar(v16):
    return v16[0]


def _chunk_of(rr):
    c = lax.shift_right_logical(rr, CSHIFT)
    return c + jnp.where(rr >= TAIL1, jnp.int32(1), jnp.int32(0))


def _route_kernel(ni_hbm, nj_hbm, tab_t_hbm, tab_tail_hbm, zbuf_hbm,
                  nodes_v, chunk0, chunk1, wl_r, wl_d, rl_r, rl_d,
                  zst, zix, sem, csem):
    wid = lax.axis_index("s") * NC + lax.axis_index("c")
    lane = lax.iota(jnp.int32, L)

    pltpu.sync_copy(ni_hbm, nodes_v.at[pl.ds(0, BATCH)])
    pltpu.sync_copy(nj_hbm, nodes_v.at[pl.ds(BATCH, BATCH)])

    def wl_body(v, off):
        rr = nodes_v[pl.ds(v * L, L)]
        m = (_chunk_of(rr) & (NW - 1)) == wid
        cnt = _scalar(plsc.all_reduce_population_count(m))
        plsc.store_compressed(wl_r.at[pl.ds(off, L)], rr, mask=m)
        plsc.store_compressed(wl_d.at[pl.ds(off, L)], v * L + lane, mask=m)
        return off + cnt

    n_w = lax.fori_loop(0, NENT // L, wl_body, jnp.int32(0))
    nv_w = (n_w + (L - 1)) // L

    def do_round(c, base, size, chunk_v):
        def rl_body(v, off):
            e = v * L + lane
            rr = wl_r[pl.ds(v * L, L)]
            dd = wl_d[pl.ds(v * L, L)]
            m = (_chunk_of(rr) == c) & (e < n_w)
            cnt = _scalar(plsc.all_reduce_population_count(m))
            plsc.store_compressed(rl_r.at[pl.ds(off, L)], rr - base, mask=m)
            plsc.store_compressed(rl_d.at[pl.ds(off, L)], dd, mask=m)
            return off + cnt

        n_r = lax.fori_loop(0, nv_w, rl_body, jnp.int32(0))

        def fill(e0):
            ng = lax.max(jnp.int32(0),
                         lax.min(jnp.int32(ZGROUPS),
                                 (n_r - e0 + (L - 1)) // L))

            def clear_body(g, carry):
                zix[pl.ds(g * L, L)] = jnp.full((L,), -1, jnp.int32)
                return carry

            lax.fori_loop(ng, ZGROUPS, clear_body, 0)

            def fill_body(g, carry):
                e = e0 + g * L
                lr = rl_r[pl.ds(e, L)]
                lr = lax.max(jnp.int32(0),
                             lax.min(lr, jnp.int32(size - 1)))
                ld = rl_d[pl.ds(e, L)]
                ld = jnp.where(e + lane < n_r, ld, jnp.int32(-1))
                zix[pl.ds(g * L, L)] = ld
                row = g * L + lane
                for d in range(DIM):
                    col = jnp.full((L,), d, jnp.int32)
                    v = plsc.load_gather(chunk_v, [col, lr])
                    plsc.store_scatter(zst, [row, col], v)
                return carry

            lax.fori_loop(0, ng, fill_body, 0)

        def burst(p, carry):
            e0 = p * ZPAD
            fill(e0)
            pltpu.async_copy(
                zst, zbuf_hbm.at[plsc.Indices(zix, ignored_value=-1)],
                sem).wait()
            return carry

        nb = (n_r + (ZPAD - 1)) // ZPAD
        lax.fori_loop(0, nb, burst, jnp.int32(0))

    def issue(t, buf):
        base = (wid + t * NW) * C
        return pltpu.async_copy(tab_t_hbm.at[:, pl.ds(base, C)], buf, csem)

    def drain(buf):
        pltpu.make_async_copy(tab_t_hbm.at[:, pl.ds(0, C)], buf, csem).wait()

    issue(0, chunk0)

    def pair_body(t2, carry):
        t = t2 * 2
        drain(chunk0)
        issue(t + 1, chunk1)
        do_round(wid + t * NW, (wid + t * NW) * C, C, chunk0)

        drain(chunk1)

        @pl.when(t2 < FULL_ROUNDS // 2 - 1)
        def _():
            issue(t + 2, chunk0)

        do_round(wid + (t + 1) * NW, (wid + (t + 1) * NW) * C, C, chunk1)
        return carry

    lax.fori_loop(0, FULL_ROUNDS // 2, pair_body, jnp.int32(0))

    last_c = wid + FULL_ROUNDS * NW

    @pl.when(wid < NFULL - FULL_ROUNDS * NW)
    def _():
        pltpu.sync_copy(tab_t_hbm.at[:, pl.ds(last_c * C, C)], chunk0)
        do_round(last_c, last_c * C, C, chunk0)

    @pl.when(wid == NFULL - FULL_ROUNDS * NW)
    def _():
        pltpu.sync_copy(tab_t_hbm.at[:, pl.ds(TAIL0, 512)],
                        chunk0.at[:, pl.ds(0, 512)])
        do_round(last_c, jnp.int32(TAIL0), 512, chunk0)

    @pl.when(wid == NFULL - FULL_ROUNDS * NW + 1)
    def _():
        pltpu.sync_copy(tab_tail_hbm, chunk0.at[:, pl.ds(0, 128)])
        do_round(last_c, jnp.int32(TAIL1), 128, chunk0)


def _norm_kernel(zbuf_hbm, beta_hbm, out_hbm, zi_v, zj_v, out_v, beta_v):
    wid = lax.axis_index("s") * NC + lax.axis_index("c")
    lane = lax.iota(jnp.int32, L)
    pltpu.sync_copy(beta_hbm, beta_v)
    beta_vec = beta_v[...]
    bpw = BATCH // NW
    slab = 128

    def slab_body(s, carry):
        b0 = wid * bpw + s * slab
        pltpu.sync_copy(zbuf_hbm.at[pl.ds(b0, slab)], zi_v)
        pltpu.sync_copy(zbuf_hbm.at[pl.ds(BATCH + b0, slab)], zj_v)

        def group_body(g, carry2):
            row = g * L + lane
            acc = jnp.zeros((L,), jnp.float32)
            for d in range(DIM):
                col = jnp.full((L,), d, jnp.int32)
                vi = plsc.load_gather(zi_v, [row, col])
                vj = plsc.load_gather(zj_v, [row, col])
                diff = vi - vj
                acc = acc + diff * diff
            norm = acc * _rsqrt16(acc)
            out_v[pl.ds(s * slab + g * L, L)] = beta_vec - norm
            return carry2

        lax.fori_loop(0, slab // L, group_body, 0)
        return carry

    lax.fori_loop(0, bpw // slab, slab_body, 0)
    pltpu.sync_copy(out_v, out_hbm.at[pl.ds(wid * bpw, bpw)])


def kernel(node_i, node_j, table, beta):
    ni = node_i.astype(jnp.int32)
    nj = node_j.astype(jnp.int32)
    beta_vec = jnp.full((L,), beta, jnp.float32)
    tab_t = table.T
    tab_tail = lax.pad(lax.slice(tab_t, (0, TAIL1), (DIM, ROWS)),
                       jnp.float32(0), ((0, 0, 0), (0, 64, 0)))

    mesh = plsc.VectorSubcoreMesh(core_axis_name="c", subcore_axis_name="s")
    params = pltpu.CompilerParams(
        needs_layout_passes=False, use_tc_tiling_on_sc=True)

    route = pl.kernel(
        _route_kernel,
        mesh=mesh,
        compiler_params=params,
        out_type=jax.ShapeDtypeStruct((NENT, ZPAD), jnp.float32),
        scratch_types=[
            pltpu.VMEM((NENT,), jnp.int32),
            pltpu.VMEM((DIM, C), jnp.float32),
            pltpu.VMEM((DIM, C), jnp.float32),
            pltpu.VMEM((WL_CAP,), jnp.int32),
            pltpu.VMEM((WL_CAP,), jnp.int32),
            pltpu.VMEM((RL_CAP,), jnp.int32),
            pltpu.VMEM((RL_CAP,), jnp.int32),
            pltpu.VMEM((ZPAD, ZPAD), jnp.float32),
            pltpu.VMEM((ZPAD,), jnp.int32),
            pltpu.SemaphoreType.DMA,
            pltpu.SemaphoreType.DMA,
        ],
    )
    zbuf = route(ni, nj, tab_t, tab_tail)

    norm = pl.kernel(
        _norm_kernel,
        mesh=mesh,
        compiler_params=params,
        out_type=jax.ShapeDtypeStruct((BATCH,), jnp.float32),
        scratch_types=[
            pltpu.VMEM((128, ZPAD), jnp.float32),
            pltpu.VMEM((128, ZPAD), jnp.float32),
            pltpu.VMEM((BATCH // NW,), jnp.float32),
            pltpu.VMEM((L,), jnp.float32),
        ],
    )
    return norm(zbuf, beta_vec)


# speedup vs baseline: 3.5814x; 1.0876x over previous
import jax
import jax.numpy as jnp
from jax import lax
from jax.experimental import pallas as pl
from jax.experimental.pallas import tpu as pltpu
from jax.experimental.pallas import tpu_sc as plsc

NC = 2
NS = 16
L = 16
NW = NC * NS

BATCH = 16384
DIM = 32
ROWS = 1_000_000
NENT = 2 * BATCH

C = 1024
CSHIFT = 10
NFULL = ROWS // C
TAIL0 = NFULL * C
TAIL1 = TAIL0 + 512
FULL_ROUNDS = NFULL // NW

WL_CAP = 2048
RL_CAP = 1024
ZPAD = 128
ZGROUPS = ZPAD // L


def _rsqrt16(x):
    i = plsc.bitcast(x, jnp.int32)
    i = jnp.int32(0x5F3759DF) - lax.shift_right_logical(i, 1)
    y = plsc.bitcast(i, jnp.float32)
    half_x = x * 0.5
    for _ in range(3):
        y = y * (1.5 - half_x * y * y)
    return y


def _scalar(v16):
    return v16[0]


def _chunk_of(rr):
    c = lax.shift_right_logical(rr, CSHIFT)
    return c + jnp.where(rr >= TAIL1, jnp.int32(1), jnp.int32(0))


def _route_kernel(ni_hbm, nj_hbm, tab_t_hbm, tab_tail_hbm, zbuf_hbm,
                  nodes_v, chunk0, chunk1, wl_r, wl_d, rl_r, rl_d,
                  zst, zix, sem, csem):
    wid = lax.axis_index("s") * NC + lax.axis_index("c")
    lane = lax.iota(jnp.int32, L)

    pltpu.sync_copy(ni_hbm, nodes_v.at[pl.ds(0, BATCH)])
    pltpu.sync_copy(nj_hbm, nodes_v.at[pl.ds(BATCH, BATCH)])

    def wl_body(v, offs):
        off_i, off_j = offs
        rr_i = nodes_v[pl.ds(v * L, L)]
        rr_j = nodes_v[pl.ds(BATCH + v * L, L)]
        m_i = (_chunk_of(rr_i) & (NW - 1)) == wid
        m_j = (_chunk_of(rr_j) & (NW - 1)) == wid
        cnt_i = _scalar(plsc.all_reduce_population_count(m_i))
        cnt_j = _scalar(plsc.all_reduce_population_count(m_j))
        plsc.store_compressed(wl_r.at[pl.ds(off_i, L)], rr_i, mask=m_i)
        plsc.store_compressed(wl_d.at[pl.ds(off_i, L)], v * L + lane,
                              mask=m_i)
        plsc.store_compressed(wl_r.at[pl.ds(WL_CAP + off_j, L)], rr_j,
                              mask=m_j)
        plsc.store_compressed(wl_d.at[pl.ds(WL_CAP + off_j, L)],
                              BATCH + v * L + lane, mask=m_j)
        return off_i + cnt_i, off_j + cnt_j

    n_wi, n_wj = lax.fori_loop(0, BATCH // L, wl_body,
                               (jnp.int32(0), jnp.int32(0)))
    nv_wi = (n_wi + (L - 1)) // L
    nv_wj = (n_wj + (L - 1)) // L
    JBOUND = 500_000 // C

    def do_round(c, base, size, chunk_v):
        def rl_body(side_base, n_side):
            def body(v, off):
                e = v * L + lane
                rr = wl_r[pl.ds(side_base + v * L, L)]
                dd = wl_d[pl.ds(side_base + v * L, L)]
                m = (_chunk_of(rr) == c) & (e < n_side)
                cnt = _scalar(plsc.all_reduce_population_count(m))
                plsc.store_compressed(rl_r.at[pl.ds(off, L)], rr - base,
                                      mask=m)
                plsc.store_compressed(rl_d.at[pl.ds(off, L)], dd, mask=m)
                return off + cnt
            return body

        nv_i_eff = jnp.where(c <= JBOUND, nv_wi, 0)
        nv_j_eff = jnp.where(c >= JBOUND, nv_wj, 0)
        n_r = lax.fori_loop(0, nv_i_eff, rl_body(0, n_wi), jnp.int32(0))
        n_r = lax.fori_loop(0, nv_j_eff, rl_body(WL_CAP, n_wj), n_r)

        def fill(e0):
            ng = lax.max(jnp.int32(0),
                         lax.min(jnp.int32(ZGROUPS),
                                 (n_r - e0 + (L - 1)) // L))

            def clear_body(g, carry):
                zix[pl.ds(g * L, L)] = jnp.full((L,), -1, jnp.int32)
                return carry

            lax.fori_loop(ng, ZGROUPS, clear_body, 0)

            def fill_body(g, carry):
                e = e0 + g * L
                lr = rl_r[pl.ds(e, L)]
                lr = lax.max(jnp.int32(0),
                             lax.min(lr, jnp.int32(size - 1)))
                ld = rl_d[pl.ds(e, L)]
                ld = jnp.where(e + lane < n_r, ld, jnp.int32(-1))
                zix[pl.ds(g * L, L)] = ld
                row = g * L + lane
                for d in range(DIM):
                    col = jnp.full((L,), d, jnp.int32)
                    v = plsc.load_gather(chunk_v, [col, lr])
                    plsc.store_scatter(zst, [row, col], v)
                return carry

            lax.fori_loop(0, ng, fill_body, 0)

        def burst(p, carry):
            e0 = p * ZPAD
            fill(e0)
            pltpu.async_copy(
                zst, zbuf_hbm.at[plsc.Indices(zix, ignored_value=-1)],
                sem).wait()
            return carry

        nb = (n_r + (ZPAD - 1)) // ZPAD
        lax.fori_loop(0, nb, burst, jnp.int32(0))

    def issue(t, buf):
        base = (wid + t * NW) * C
        return pltpu.async_copy(tab_t_hbm.at[:, pl.ds(base, C)], buf, csem)

    def drain(buf):
        pltpu.make_async_copy(tab_t_hbm.at[:, pl.ds(0, C)], buf, csem).wait()

    issue(0, chunk0)

    def pair_body(t2, carry):
        t = t2 * 2
        drain(chunk0)
        issue(t + 1, chunk1)
        do_round(wid + t * NW, (wid + t * NW) * C, C, chunk0)

        drain(chunk1)

        @pl.when(t2 < FULL_ROUNDS // 2 - 1)
        def _():
            issue(t + 2, chunk0)

        do_round(wid + (t + 1) * NW, (wid + (t + 1) * NW) * C, C, chunk1)
        return carry

    lax.fori_loop(0, FULL_ROUNDS // 2, pair_body, jnp.int32(0))

    last_c = wid + FULL_ROUNDS * NW

    @pl.when(wid < NFULL - FULL_ROUNDS * NW)
    def _():
        pltpu.sync_copy(tab_t_hbm.at[:, pl.ds(last_c * C, C)], chunk0)
        do_round(last_c, last_c * C, C, chunk0)

    @pl.when(wid == NFULL - FULL_ROUNDS * NW)
    def _():
        pltpu.sync_copy(tab_t_hbm.at[:, pl.ds(TAIL0, 512)],
                        chunk0.at[:, pl.ds(0, 512)])
        do_round(last_c, jnp.int32(TAIL0), 512, chunk0)

    @pl.when(wid == NFULL - FULL_ROUNDS * NW + 1)
    def _():
        pltpu.sync_copy(tab_tail_hbm, chunk0.at[:, pl.ds(0, 128)])
        do_round(last_c, jnp.int32(TAIL1), 128, chunk0)


def _norm_kernel(zbuf_hbm, beta_hbm, out_hbm, zi_v, zj_v, out_v, beta_v):
    wid = lax.axis_index("s") * NC + lax.axis_index("c")
    lane = lax.iota(jnp.int32, L)
    pltpu.sync_copy(beta_hbm, beta_v)
    beta_vec = beta_v[...]
    bpw = BATCH // NW
    slab = 128

    def slab_body(s, carry):
        b0 = wid * bpw + s * slab
        pltpu.sync_copy(zbuf_hbm.at[pl.ds(b0, slab)], zi_v)
        pltpu.sync_copy(zbuf_hbm.at[pl.ds(BATCH + b0, slab)], zj_v)

        def group_body(g, carry2):
            row = g * L + lane
            acc = jnp.zeros((L,), jnp.float32)
            for d in range(DIM):
                col = jnp.full((L,), d, jnp.int32)
                vi = plsc.load_gather(zi_v, [row, col])
                vj = plsc.load_gather(zj_v, [row, col])
                diff = vi - vj
                acc = acc + diff * diff
            norm = acc * _rsqrt16(acc)
            out_v[pl.ds(s * slab + g * L, L)] = beta_vec - norm
            return carry2

        lax.fori_loop(0, slab // L, group_body, 0)
        return carry

    lax.fori_loop(0, bpw // slab, slab_body, 0)
    pltpu.sync_copy(out_v, out_hbm.at[pl.ds(wid * bpw, bpw)])


def kernel(node_i, node_j, table, beta):
    ni = node_i.astype(jnp.int32)
    nj = node_j.astype(jnp.int32)
    beta_vec = jnp.full((L,), beta, jnp.float32)
    tab_t = table.T
    tab_tail = lax.pad(lax.slice(tab_t, (0, TAIL1), (DIM, ROWS)),
                       jnp.float32(0), ((0, 0, 0), (0, 64, 0)))

    mesh = plsc.VectorSubcoreMesh(core_axis_name="c", subcore_axis_name="s")
    params = pltpu.CompilerParams(
        needs_layout_passes=False, use_tc_tiling_on_sc=True)

    route = pl.kernel(
        _route_kernel,
        mesh=mesh,
        compiler_params=params,
        out_type=jax.ShapeDtypeStruct((NENT, ZPAD), jnp.float32),
        scratch_types=[
            pltpu.VMEM((NENT,), jnp.int32),
            pltpu.VMEM((DIM, C), jnp.float32),
            pltpu.VMEM((DIM, C), jnp.float32),
            pltpu.VMEM((2 * WL_CAP,), jnp.int32),
            pltpu.VMEM((2 * WL_CAP,), jnp.int32),
            pltpu.VMEM((RL_CAP,), jnp.int32),
            pltpu.VMEM((RL_CAP,), jnp.int32),
            pltpu.VMEM((ZPAD, ZPAD), jnp.float32),
            pltpu.VMEM((ZPAD,), jnp.int32),
            pltpu.SemaphoreType.DMA,
            pltpu.SemaphoreType.DMA,
        ],
    )
    zbuf = route(ni, nj, tab_t, tab_tail)

    norm = pl.kernel(
        _norm_kernel,
        mesh=mesh,
        compiler_params=params,
        out_type=jax.ShapeDtypeStruct((BATCH,), jnp.float32),
        scratch_types=[
            pltpu.VMEM((128, ZPAD), jnp.float32),
            pltpu.VMEM((128, ZPAD), jnp.float32),
            pltpu.VMEM((BATCH // NW,), jnp.float32),
            pltpu.VMEM((L,), jnp.float32),
        ],
    )
    return norm(zbuf, beta_vec)


# speedup vs baseline: 4.0783x; 1.1388x over previous
import jax
import jax.numpy as jnp
from jax import lax
from jax.experimental import pallas as pl
from jax.experimental.pallas import tpu as pltpu
from jax.experimental.pallas import tpu_sc as plsc

NC = 2
NS = 16
L = 16
NW = NC * NS

BATCH = 16384
DIM = 32
ROWS = 1_000_000
NENT = 2 * BATCH

C = 1024
CSHIFT = 10
NFULL = ROWS // C
TAIL0 = NFULL * C
TAIL1 = TAIL0 + 512
FULL_ROUNDS = NFULL // NW

WL_CAP = 2048
RL_CAP = 1024
ZPAD = 128
ZGROUPS = ZPAD // L


def _rsqrt16(x):
    i = plsc.bitcast(x, jnp.int32)
    i = jnp.int32(0x5F3759DF) - lax.shift_right_logical(i, 1)
    y = plsc.bitcast(i, jnp.float32)
    half_x = x * 0.5
    for _ in range(3):
        y = y * (1.5 - half_x * y * y)
    return y


def _scalar(v16):
    return v16[0]


def _chunk_of(rr):
    c = lax.shift_right_logical(rr, CSHIFT)
    return c + jnp.where(rr >= TAIL1, jnp.int32(1), jnp.int32(0))


def _route_kernel(ni_hbm, nj_hbm, tab_t_hbm, tab_tail_hbm, zbuf_hbm,
                  nodes_v, chunk0, chunk1, wl_r, wl_d, rl_r, rl_d,
                  zst, zix, sem, csem):
    wid = lax.axis_index("s") * NC + lax.axis_index("c")
    lane = lax.iota(jnp.int32, L)

    pltpu.sync_copy(ni_hbm, nodes_v.at[pl.ds(0, BATCH)])
    pltpu.sync_copy(nj_hbm, nodes_v.at[pl.ds(BATCH, BATCH)])

    def wl_body(v, offs):
        off_i, off_j = offs
        rr_i = nodes_v[pl.ds(v * L, L)]
        rr_j = nodes_v[pl.ds(BATCH + v * L, L)]
        m_i = (_chunk_of(rr_i) & (NW - 1)) == wid
        m_j = (_chunk_of(rr_j) & (NW - 1)) == wid
        cnt_i = _scalar(plsc.all_reduce_population_count(m_i))
        cnt_j = _scalar(plsc.all_reduce_population_count(m_j))
        plsc.store_compressed(wl_r.at[pl.ds(off_i, L)], rr_i, mask=m_i)
        plsc.store_compressed(wl_d.at[pl.ds(off_i, L)], v * L + lane,
                              mask=m_i)
        plsc.store_compressed(wl_r.at[pl.ds(WL_CAP + off_j, L)], rr_j,
                              mask=m_j)
        plsc.store_compressed(wl_d.at[pl.ds(WL_CAP + off_j, L)],
                              BATCH + v * L + lane, mask=m_j)
        return off_i + cnt_i, off_j + cnt_j

    n_wi, n_wj = lax.fori_loop(0, BATCH // L, wl_body,
                               (jnp.int32(0), jnp.int32(0)))
    nv_wi = (n_wi + (L - 1)) // L
    nv_wj = (n_wj + (L - 1)) // L
    JBOUND = 500_000 // C

    def do_round(c, base, size, chunk_v):
        def rl_body(side_base, n_side):
            def body(v, off):
                e = v * L + lane
                rr = wl_r[pl.ds(side_base + v * L, L)]
                dd = wl_d[pl.ds(side_base + v * L, L)]
                m = (_chunk_of(rr) == c) & (e < n_side)
                cnt = _scalar(plsc.all_reduce_population_count(m))
                plsc.store_compressed(rl_r.at[pl.ds(off, L)], rr - base,
                                      mask=m)
                plsc.store_compressed(rl_d.at[pl.ds(off, L)], dd, mask=m)
                return off + cnt
            return body

        nv_i_eff = jnp.where(c <= JBOUND, nv_wi, 0)
        nv_j_eff = jnp.where(c >= JBOUND, nv_wj, 0)
        n_r = lax.fori_loop(0, nv_i_eff, rl_body(0, n_wi), jnp.int32(0))
        n_r = lax.fori_loop(0, nv_j_eff, rl_body(WL_CAP, n_wj), n_r)

        def fill(e0):
            ng = lax.max(jnp.int32(0),
                         lax.min(jnp.int32(ZGROUPS),
                                 (n_r - e0 + (L - 1)) // L))

            def clear_body(g, carry):
                zix[pl.ds(g * L, L)] = jnp.full((L,), -1, jnp.int32)
                return carry

            lax.fori_loop(ng, ZGROUPS, clear_body, 0)

            def fill_body(g, carry):
                e = e0 + g * L
                lr = rl_r[pl.ds(e, L)]
                lr = lax.max(jnp.int32(0),
                             lax.min(lr, jnp.int32(size - 1)))
                ld = rl_d[pl.ds(e, L)]
                ld = jnp.where(e + lane < n_r, ld, jnp.int32(-1))
                zix[pl.ds(g * L, L)] = ld
                row = g * L + lane
                for d in range(DIM):
                    col = jnp.full((L,), d, jnp.int32)
                    v = plsc.load_gather(chunk_v, [col, lr])
                    plsc.store_scatter(zst, [row, col], v)
                return carry

            lax.fori_loop(0, ng, fill_body, 0)

        def burst(p, carry):
            e0 = p * ZPAD
            fill(e0)
            pltpu.async_copy(
                zst, zbuf_hbm.at[plsc.Indices(zix, ignored_value=-1)],
                sem).wait()
            return carry

        nb = (n_r + (ZPAD - 1)) // ZPAD
        lax.fori_loop(0, nb, burst, jnp.int32(0))

    def issue(t, buf):
        base = (wid + t * NW) * C
        return pltpu.async_copy(tab_t_hbm.at[:, pl.ds(base, C)], buf, csem)

    def drain(buf):
        pltpu.make_async_copy(tab_t_hbm.at[:, pl.ds(0, C)], buf, csem).wait()

    issue(0, chunk0)

    def pair_body(t2, carry):
        t = t2 * 2
        drain(chunk0)
        issue(t + 1, chunk1)
        do_round(wid + t * NW, (wid + t * NW) * C, C, chunk0)

        drain(chunk1)

        @pl.when(t2 < FULL_ROUNDS // 2 - 1)
        def _():
            issue(t + 2, chunk0)

        do_round(wid + (t + 1) * NW, (wid + (t + 1) * NW) * C, C, chunk1)
        return carry

    lax.fori_loop(0, FULL_ROUNDS // 2, pair_body, jnp.int32(0))

    last_c = wid + FULL_ROUNDS * NW

    @pl.when(wid < NFULL - FULL_ROUNDS * NW)
    def _():
        pltpu.sync_copy(tab_t_hbm.at[:, pl.ds(last_c * C, C)], chunk0)
        do_round(last_c, last_c * C, C, chunk0)

    @pl.when(wid == NFULL - FULL_ROUNDS * NW)
    def _():
        pltpu.sync_copy(tab_t_hbm.at[:, pl.ds(TAIL0, 512)],
                        chunk0.at[:, pl.ds(0, 512)])
        do_round(last_c, jnp.int32(TAIL0), 512, chunk0)

    @pl.when(wid == NFULL - FULL_ROUNDS * NW + 1)
    def _():
        pltpu.sync_copy(tab_tail_hbm, chunk0.at[:, pl.ds(0, 128)])
        do_round(last_c, jnp.int32(TAIL1), 128, chunk0)


def _norm_tc_kernel(zi_ref, zj_ref, beta_ref, out_ref):
    zi = zi_ref[:, :DIM]
    zj = zj_ref[:, :DIM]
    d = zi - zj
    ss = jnp.sum(d * d, axis=1)
    out_ref[...] = beta_ref[0] - jnp.sqrt(ss)


def _norm_kernel(zbuf_hbm, beta_hbm, out_hbm, zi_v, zj_v, out_v, beta_v):
    wid = lax.axis_index("s") * NC + lax.axis_index("c")
    lane = lax.iota(jnp.int32, L)
    pltpu.sync_copy(beta_hbm, beta_v)
    beta_vec = beta_v[...]
    bpw = BATCH // NW
    slab = 128

    def slab_body(s, carry):
        b0 = wid * bpw + s * slab
        pltpu.sync_copy(zbuf_hbm.at[pl.ds(b0, slab)], zi_v)
        pltpu.sync_copy(zbuf_hbm.at[pl.ds(BATCH + b0, slab)], zj_v)

        def group_body(g, carry2):
            row = g * L + lane
            acc = jnp.zeros((L,), jnp.float32)
            for d in range(DIM):
                col = jnp.full((L,), d, jnp.int32)
                vi = plsc.load_gather(zi_v, [row, col])
                vj = plsc.load_gather(zj_v, [row, col])
                diff = vi - vj
                acc = acc + diff * diff
            norm = acc * _rsqrt16(acc)
            out_v[pl.ds(s * slab + g * L, L)] = beta_vec - norm
            return carry2

        lax.fori_loop(0, slab // L, group_body, 0)
        return carry

    lax.fori_loop(0, bpw // slab, slab_body, 0)
    pltpu.sync_copy(out_v, out_hbm.at[pl.ds(wid * bpw, bpw)])


def kernel(node_i, node_j, table, beta):
    ni = node_i.astype(jnp.int32)
    nj = node_j.astype(jnp.int32)
    beta_vec = jnp.full((L,), beta, jnp.float32)
    tab_t = table.T
    tab_tail = lax.pad(lax.slice(tab_t, (0, TAIL1), (DIM, ROWS)),
                       jnp.float32(0), ((0, 0, 0), (0, 64, 0)))

    mesh = plsc.VectorSubcoreMesh(core_axis_name="c", subcore_axis_name="s")
    params = pltpu.CompilerParams(
        needs_layout_passes=False, use_tc_tiling_on_sc=True)

    route = pl.kernel(
        _route_kernel,
        mesh=mesh,
        compiler_params=params,
        out_type=jax.ShapeDtypeStruct((NENT, ZPAD), jnp.float32),
        scratch_types=[
            pltpu.VMEM((NENT,), jnp.int32),
            pltpu.VMEM((DIM, C), jnp.float32),
            pltpu.VMEM((DIM, C), jnp.float32),
            pltpu.VMEM((2 * WL_CAP,), jnp.int32),
            pltpu.VMEM((2 * WL_CAP,), jnp.int32),
            pltpu.VMEM((RL_CAP,), jnp.int32),
            pltpu.VMEM((RL_CAP,), jnp.int32),
            pltpu.VMEM((ZPAD, ZPAD), jnp.float32),
            pltpu.VMEM((ZPAD,), jnp.int32),
            pltpu.SemaphoreType.DMA,
            pltpu.SemaphoreType.DMA,
        ],
    )
    zbuf = route(ni, nj, tab_t, tab_tail)

    G = 2048
    return pl.pallas_call(
        _norm_tc_kernel,
        grid=(BATCH // G,),
        in_specs=[
            pl.BlockSpec((G, ZPAD), lambda i: (i, 0)),
            pl.BlockSpec((G, ZPAD), lambda i: (i + BATCH // G, 0)),
            pl.BlockSpec(memory_space=pltpu.SMEM),
        ],
        out_specs=pl.BlockSpec((G,), lambda i: (i,)),
        out_shape=jax.ShapeDtypeStruct((BATCH,), jnp.float32),
    )(zbuf, zbuf, jnp.full((1,), beta, jnp.float32))


# speedup vs baseline: 4.1790x; 1.0247x over previous
import jax
import jax.numpy as jnp
from jax import lax
from jax.experimental import pallas as pl
from jax.experimental.pallas import tpu as pltpu
from jax.experimental.pallas import tpu_sc as plsc

NC = 2
NS = 16
L = 16
NW = NC * NS

BATCH = 16384
DIM = 32
ROWS = 1_000_000
NENT = 2 * BATCH

C = 1024
CSHIFT = 10
NFULL = ROWS // C
TAIL0 = NFULL * C
TAIL1 = TAIL0 + 512
FULL_ROUNDS = NFULL // NW

WL_CAP = 2048
RL_CAP = 1024
ZPAD = 128
ZGROUPS = ZPAD // L


def _rsqrt16(x):
    i = plsc.bitcast(x, jnp.int32)
    i = jnp.int32(0x5F3759DF) - lax.shift_right_logical(i, 1)
    y = plsc.bitcast(i, jnp.float32)
    half_x = x * 0.5
    for _ in range(3):
        y = y * (1.5 - half_x * y * y)
    return y


def _scalar(v16):
    return v16[0]


def _chunk_of(rr):
    c = lax.shift_right_logical(rr, CSHIFT)
    return c + jnp.where(rr >= TAIL1, jnp.int32(1), jnp.int32(0))


def _route_kernel(ni_hbm, nj_hbm, tab_t_hbm, tab_tail_hbm, zbuf_hbm,
                  nodes_v, chunk0, chunk1, wl_r, wl_d, rl_r, rl_d,
                  zst, zix, sem, csem):
    wid = lax.axis_index("s") * NC + lax.axis_index("c")
    lane = lax.iota(jnp.int32, L)

    pltpu.sync_copy(ni_hbm, nodes_v.at[pl.ds(0, BATCH)])
    pltpu.sync_copy(nj_hbm, nodes_v.at[pl.ds(BATCH, BATCH)])

    def wl_body(v, offs):
        off_i, off_j = offs
        rr_i = nodes_v[pl.ds(v * L, L)]
        rr_j = nodes_v[pl.ds(BATCH + v * L, L)]
        m_i = (_chunk_of(rr_i) & (NW - 1)) == wid
        m_j = (_chunk_of(rr_j) & (NW - 1)) == wid
        cnt_i = _scalar(plsc.all_reduce_population_count(m_i))
        cnt_j = _scalar(plsc.all_reduce_population_count(m_j))
        plsc.store_compressed(wl_r.at[pl.ds(off_i, L)], rr_i, mask=m_i)
        plsc.store_compressed(wl_d.at[pl.ds(off_i, L)], v * L + lane,
                              mask=m_i)
        plsc.store_compressed(wl_r.at[pl.ds(WL_CAP + off_j, L)], rr_j,
                              mask=m_j)
        plsc.store_compressed(wl_d.at[pl.ds(WL_CAP + off_j, L)],
                              BATCH + v * L + lane, mask=m_j)
        return off_i + cnt_i, off_j + cnt_j

    n_wi, n_wj = lax.fori_loop(0, BATCH // L, wl_body,
                               (jnp.int32(0), jnp.int32(0)))
    nv_wi = (n_wi + (L - 1)) // L
    nv_wj = (n_wj + (L - 1)) // L
    JBOUND = 500_000 // C

    def clear_zix():
        for g in range(ZGROUPS):
            zix[pl.ds(g * L, L)] = jnp.full((L,), -1, jnp.int32)

    def flush():
        pltpu.async_copy(
            zst, zbuf_hbm.at[plsc.Indices(zix, ignored_value=-1)],
            sem).wait()
        clear_zix()

    def do_round(c, base, size, chunk_v, zfill):
        def rl_body(side_base, n_side):
            def body(v, off):
                e = v * L + lane
                rr = wl_r[pl.ds(side_base + v * L, L)]
                dd = wl_d[pl.ds(side_base + v * L, L)]
                m = (_chunk_of(rr) == c) & (e < n_side)
                cnt = _scalar(plsc.all_reduce_population_count(m))
                plsc.store_compressed(rl_r.at[pl.ds(off, L)], rr - base,
                                      mask=m)
                plsc.store_compressed(rl_d.at[pl.ds(off, L)], dd, mask=m)
                return off + cnt
            return body

        nv_i_eff = jnp.where(c <= JBOUND, nv_wi, 0)
        nv_j_eff = jnp.where(c >= JBOUND, nv_wj, 0)
        n_r = lax.fori_loop(0, nv_i_eff, rl_body(0, n_wi), jnp.int32(0))
        n_r = lax.fori_loop(0, nv_j_eff, rl_body(WL_CAP, n_wj), n_r)

        def group_body(g, zf):
            e = g * L
            lr = rl_r[pl.ds(e, L)]
            lr = lax.max(jnp.int32(0),
                         lax.min(lr, jnp.int32(size - 1)))
            ld = rl_d[pl.ds(e, L)]
            ld = jnp.where(e + lane < n_r, ld, jnp.int32(-1))
            zix[pl.ds(zf, L)] = ld
            row = zf + lane
            for d in range(DIM):
                col = jnp.full((L,), d, jnp.int32)
                v = plsc.load_gather(chunk_v, [col, lr])
                plsc.store_scatter(zst, [row, col], v)
            zf = zf + L

            @pl.when(zf == ZPAD)
            def _():
                flush()

            return jnp.where(zf == ZPAD, jnp.int32(0), zf)

        ng = (n_r + (L - 1)) // L
        return lax.fori_loop(0, ng, group_body, zfill)

    def issue(t, buf):
        base = (wid + t * NW) * C
        return pltpu.async_copy(tab_t_hbm.at[:, pl.ds(base, C)], buf, csem)

    def drain(buf):
        pltpu.make_async_copy(tab_t_hbm.at[:, pl.ds(0, C)], buf, csem).wait()

    clear_zix()
    issue(0, chunk0)

    def pair_body(t2, zfill):
        t = t2 * 2
        drain(chunk0)
        issue(t + 1, chunk1)
        zfill = do_round(wid + t * NW, (wid + t * NW) * C, C, chunk0, zfill)

        drain(chunk1)

        @pl.when(t2 < FULL_ROUNDS // 2 - 1)
        def _():
            issue(t + 2, chunk0)

        return do_round(wid + (t + 1) * NW, (wid + (t + 1) * NW) * C, C,
                        chunk1, zfill)

    zfill = lax.fori_loop(0, FULL_ROUNDS // 2, pair_body, jnp.int32(0))

    last_c = wid + FULL_ROUNDS * NW

    @pl.when(wid < NFULL - FULL_ROUNDS * NW)
    def _():
        pltpu.sync_copy(tab_t_hbm.at[:, pl.ds(last_c * C, C)], chunk0)
        zf = do_round(last_c, last_c * C, C, chunk0, zfill)

        @pl.when(zf > 0)
        def _():
            flush()

    @pl.when(wid == NFULL - FULL_ROUNDS * NW)
    def _():
        pltpu.sync_copy(tab_t_hbm.at[:, pl.ds(TAIL0, 512)],
                        chunk0.at[:, pl.ds(0, 512)])
        zf = do_round(last_c, jnp.int32(TAIL0), 512, chunk0, zfill)

        @pl.when(zf > 0)
        def _():
            flush()

    @pl.when(wid == NFULL - FULL_ROUNDS * NW + 1)
    def _():
        pltpu.sync_copy(tab_tail_hbm, chunk0.at[:, pl.ds(0, 128)])
        zf = do_round(last_c, jnp.int32(TAIL1), 128, chunk0, zfill)

        @pl.when(zf > 0)
        def _():
            flush()

    @pl.when((wid >= NFULL - FULL_ROUNDS * NW + 2) & (zfill > 0))
    def _():
        flush()


def _norm_tc_kernel(zi_ref, zj_ref, beta_ref, out_ref):
    zi = zi_ref[:, :DIM]
    zj = zj_ref[:, :DIM]
    d = zi - zj
    ss = jnp.sum(d * d, axis=1)
    out_ref[...] = beta_ref[0] - jnp.sqrt(ss)


def _norm_kernel(zbuf_hbm, beta_hbm, out_hbm, zi_v, zj_v, out_v, beta_v):
    wid = lax.axis_index("s") * NC + lax.axis_index("c")
    lane = lax.iota(jnp.int32, L)
    pltpu.sync_copy(beta_hbm, beta_v)
    beta_vec = beta_v[...]
    bpw = BATCH // NW
    slab = 128

    def slab_body(s, carry):
        b0 = wid * bpw + s * slab
        pltpu.sync_copy(zbuf_hbm.at[pl.ds(b0, slab)], zi_v)
        pltpu.sync_copy(zbuf_hbm.at[pl.ds(BATCH + b0, slab)], zj_v)

        def group_body(g, carry2):
            row = g * L + lane
            acc = jnp.zeros((L,), jnp.float32)
            for d in range(DIM):
                col = jnp.full((L,), d, jnp.int32)
                vi = plsc.load_gather(zi_v, [row, col])
                vj = plsc.load_gather(zj_v, [row, col])
                diff = vi - vj
                acc = acc + diff * diff
            norm = acc * _rsqrt16(acc)
            out_v[pl.ds(s * slab + g * L, L)] = beta_vec - norm
            return carry2

        lax.fori_loop(0, slab // L, group_body, 0)
        return carry

    lax.fori_loop(0, bpw // slab, slab_body, 0)
    pltpu.sync_copy(out_v, out_hbm.at[pl.ds(wid * bpw, bpw)])


def kernel(node_i, node_j, table, beta):
    ni = node_i.astype(jnp.int32)
    nj = node_j.astype(jnp.int32)
    beta_vec = jnp.full((L,), beta, jnp.float32)
    tab_t = table.T
    tab_tail = lax.pad(lax.slice(tab_t, (0, TAIL1), (DIM, ROWS)),
                       jnp.float32(0), ((0, 0, 0), (0, 64, 0)))

    mesh = plsc.VectorSubcoreMesh(core_axis_name="c", subcore_axis_name="s")
    params = pltpu.CompilerParams(
        needs_layout_passes=False, use_tc_tiling_on_sc=True)

    route = pl.kernel(
        _route_kernel,
        mesh=mesh,
        compiler_params=params,
        out_type=jax.ShapeDtypeStruct((NENT, ZPAD), jnp.float32),
        scratch_types=[
            pltpu.VMEM((NENT,), jnp.int32),
            pltpu.VMEM((DIM, C), jnp.float32),
            pltpu.VMEM((DIM, C), jnp.float32),
            pltpu.VMEM((2 * WL_CAP,), jnp.int32),
            pltpu.VMEM((2 * WL_CAP,), jnp.int32),
            pltpu.VMEM((RL_CAP,), jnp.int32),
            pltpu.VMEM((RL_CAP,), jnp.int32),
            pltpu.VMEM((ZPAD, ZPAD), jnp.float32),
            pltpu.VMEM((ZPAD,), jnp.int32),
            pltpu.SemaphoreType.DMA,
            pltpu.SemaphoreType.DMA,
        ],
    )
    zbuf = route(ni, nj, tab_t, tab_tail)

    G = 2048
    return pl.pallas_call(
        _norm_tc_kernel,
        grid=(BATCH // G,),
        in_specs=[
            pl.BlockSpec((G, ZPAD), lambda i: (i, 0)),
            pl.BlockSpec((G, ZPAD), lambda i: (i + BATCH // G, 0)),
            pl.BlockSpec(memory_space=pltpu.SMEM),
        ],
        out_specs=pl.BlockSpec((G,), lambda i: (i,)),
        out_shape=jax.ShapeDtypeStruct((BATCH,), jnp.float32),
    )(zbuf, zbuf, jnp.full((1,), beta, jnp.float32))


# speedup vs baseline: 4.2324x; 1.0128x over previous
import jax
import jax.numpy as jnp
from jax import lax
from jax.experimental import pallas as pl
from jax.experimental.pallas import tpu as pltpu
from jax.experimental.pallas import tpu_sc as plsc

NC = 2
NS = 16
L = 16
NW = NC * NS

BATCH = 16384
DIM = 32
ROWS = 1_000_000
NENT = 2 * BATCH

C = 1024
CSHIFT = 10
NFULL = ROWS // C
TAIL0 = NFULL * C
TAIL1 = TAIL0 + 512
FULL_ROUNDS = NFULL // NW

WL_CAP = 2048
RL_CAP = 1024
ZPAD = 128
ZGROUPS = ZPAD // L


def _rsqrt16(x):
    i = plsc.bitcast(x, jnp.int32)
    i = jnp.int32(0x5F3759DF) - lax.shift_right_logical(i, 1)
    y = plsc.bitcast(i, jnp.float32)
    half_x = x * 0.5
    for _ in range(3):
        y = y * (1.5 - half_x * y * y)
    return y


def _scalar(v16):
    return v16[0]


def _chunk_of(rr):
    c = lax.shift_right_logical(rr, CSHIFT)
    return c + jnp.where(rr >= TAIL1, jnp.int32(1), jnp.int32(0))


def _route_kernel(ni_hbm, nj_hbm, tab_t_hbm, tab_tail_hbm, zbuf_hbm,
                  nodes_v, chunk0, chunk1, wl_r, wl_d, rl_r, rl_d,
                  zst, zix, sem, csem):
    wid = lax.axis_index("s") * NC + lax.axis_index("c")
    lane = lax.iota(jnp.int32, L)

    pltpu.sync_copy(ni_hbm, nodes_v.at[pl.ds(0, BATCH)])
    pltpu.sync_copy(nj_hbm, nodes_v.at[pl.ds(BATCH, BATCH)])

    pltpu.async_copy(tab_t_hbm.at[:, pl.ds(wid * C, C)], chunk0, csem)

    def wl_body(v, offs):
        off_i, off_j = offs
        rr_i = nodes_v[pl.ds(v * L, L)]
        rr_j = nodes_v[pl.ds(BATCH + v * L, L)]
        m_i = (_chunk_of(rr_i) & (NW - 1)) == wid
        m_j = (_chunk_of(rr_j) & (NW - 1)) == wid
        cnt_i = _scalar(plsc.all_reduce_population_count(m_i))
        cnt_j = _scalar(plsc.all_reduce_population_count(m_j))
        plsc.store_compressed(wl_r.at[pl.ds(off_i, L)], rr_i, mask=m_i)
        plsc.store_compressed(wl_d.at[pl.ds(off_i, L)], v * L + lane,
                              mask=m_i)
        plsc.store_compressed(wl_r.at[pl.ds(WL_CAP + off_j, L)], rr_j,
                              mask=m_j)
        plsc.store_compressed(wl_d.at[pl.ds(WL_CAP + off_j, L)],
                              BATCH + v * L + lane, mask=m_j)
        return off_i + cnt_i, off_j + cnt_j

    n_wi, n_wj = lax.fori_loop(0, BATCH // L, wl_body,
                               (jnp.int32(0), jnp.int32(0)))
    nv_wi = (n_wi + (L - 1)) // L
    nv_wj = (n_wj + (L - 1)) // L
    JBOUND = 500_000 // C

    def clear_zix():
        for g in range(ZGROUPS):
            zix[pl.ds(g * L, L)] = jnp.full((L,), -1, jnp.int32)

    def flush():
        pltpu.async_copy(
            zst, zbuf_hbm.at[plsc.Indices(zix, ignored_value=-1)],
            sem).wait()
        clear_zix()

    def do_round(c, base, size, chunk_v, zfill):
        def rl_body(side_base, n_side):
            def body(v, off):
                e = v * L + lane
                rr = wl_r[pl.ds(side_base + v * L, L)]
                dd = wl_d[pl.ds(side_base + v * L, L)]
                m = (_chunk_of(rr) == c) & (e < n_side)
                cnt = _scalar(plsc.all_reduce_population_count(m))
                plsc.store_compressed(rl_r.at[pl.ds(off, L)], rr - base,
                                      mask=m)
                plsc.store_compressed(rl_d.at[pl.ds(off, L)], dd, mask=m)
                return off + cnt
            return body

        nv_i_eff = jnp.where(c <= JBOUND, nv_wi, 0)
        nv_j_eff = jnp.where(c >= JBOUND, nv_wj, 0)
        n_r = lax.fori_loop(0, nv_i_eff, rl_body(0, n_wi), jnp.int32(0))
        n_r = lax.fori_loop(0, nv_j_eff, rl_body(WL_CAP, n_wj), n_r)

        def group_body(g, zf):
            e = g * L
            lr = rl_r[pl.ds(e, L)]
            lr = lax.max(jnp.int32(0),
                         lax.min(lr, jnp.int32(size - 1)))
            ld = rl_d[pl.ds(e, L)]
            ld = jnp.where(e + lane < n_r, ld, jnp.int32(-1))
            zix[pl.ds(zf, L)] = ld
            row = zf + lane
            for d in range(DIM):
                col = jnp.full((L,), d, jnp.int32)
                v = plsc.load_gather(chunk_v, [col, lr])
                plsc.store_scatter(zst, [row, col], v)
            zf = zf + L

            @pl.when(zf == ZPAD)
            def _():
                flush()

            return jnp.where(zf == ZPAD, jnp.int32(0), zf)

        ng = (n_r + (L - 1)) // L
        return lax.fori_loop(0, ng, group_body, zfill)

    def issue(t, buf):
        base = (wid + t * NW) * C
        return pltpu.async_copy(tab_t_hbm.at[:, pl.ds(base, C)], buf, csem)

    def drain(buf):
        pltpu.make_async_copy(tab_t_hbm.at[:, pl.ds(0, C)], buf, csem).wait()

    clear_zix()

    def pair_body(t2, zfill):
        t = t2 * 2
        drain(chunk0)
        issue(t + 1, chunk1)
        zfill = do_round(wid + t * NW, (wid + t * NW) * C, C, chunk0, zfill)

        drain(chunk1)

        @pl.when(t2 < FULL_ROUNDS // 2 - 1)
        def _():
            issue(t + 2, chunk0)

        return do_round(wid + (t + 1) * NW, (wid + (t + 1) * NW) * C, C,
                        chunk1, zfill)

    zfill = lax.fori_loop(0, FULL_ROUNDS // 2, pair_body, jnp.int32(0))

    last_c = wid + FULL_ROUNDS * NW

    @pl.when(wid < NFULL - FULL_ROUNDS * NW)
    def _():
        pltpu.sync_copy(tab_t_hbm.at[:, pl.ds(last_c * C, C)], chunk0)
        zf = do_round(last_c, last_c * C, C, chunk0, zfill)

        @pl.when(zf > 0)
        def _():
            flush()

    @pl.when(wid == NFULL - FULL_ROUNDS * NW)
    def _():
        pltpu.sync_copy(tab_t_hbm.at[:, pl.ds(TAIL0, 512)],
                        chunk0.at[:, pl.ds(0, 512)])
        zf = do_round(last_c, jnp.int32(TAIL0), 512, chunk0, zfill)

        @pl.when(zf > 0)
        def _():
            flush()

    @pl.when(wid == NFULL - FULL_ROUNDS * NW + 1)
    def _():
        pltpu.sync_copy(tab_tail_hbm, chunk0.at[:, pl.ds(0, 128)])
        zf = do_round(last_c, jnp.int32(TAIL1), 128, chunk0, zfill)

        @pl.when(zf > 0)
        def _():
            flush()

    @pl.when((wid >= NFULL - FULL_ROUNDS * NW + 2) & (zfill > 0))
    def _():
        flush()


def _norm_tc_kernel(zi_ref, zj_ref, beta_ref, out_ref):
    d = zi_ref[:, :DIM] - zj_ref[:, :DIM]
    ss = jnp.sum(d * d, axis=1)
    out_ref[...] = beta_ref[0] - jnp.sqrt(ss)


def _norm_kernel(zbuf_hbm, beta_hbm, out_hbm, zi_v, zj_v, out_v, beta_v):
    wid = lax.axis_index("s") * NC + lax.axis_index("c")
    lane = lax.iota(jnp.int32, L)
    pltpu.sync_copy(beta_hbm, beta_v)
    beta_vec = beta_v[...]
    bpw = BATCH // NW
    slab = 128

    def slab_body(s, carry):
        b0 = wid * bpw + s * slab
        pltpu.sync_copy(zbuf_hbm.at[pl.ds(b0, slab)], zi_v)
        pltpu.sync_copy(zbuf_hbm.at[pl.ds(BATCH + b0, slab)], zj_v)

        def group_body(g, carry2):
            row = g * L + lane
            acc = jnp.zeros((L,), jnp.float32)
            for d in range(DIM):
                col = jnp.full((L,), d, jnp.int32)
                vi = plsc.load_gather(zi_v, [row, col])
                vj = plsc.load_gather(zj_v, [row, col])
                diff = vi - vj
                acc = acc + diff * diff
            norm = acc * _rsqrt16(acc)
            out_v[pl.ds(s * slab + g * L, L)] = beta_vec - norm
            return carry2

        lax.fori_loop(0, slab // L, group_body, 0)
        return carry

    lax.fori_loop(0, bpw // slab, slab_body, 0)
    pltpu.sync_copy(out_v, out_hbm.at[pl.ds(wid * bpw, bpw)])


def kernel(node_i, node_j, table, beta):
    ni = node_i.astype(jnp.int32)
    nj = node_j.astype(jnp.int32)
    beta_vec = jnp.full((L,), beta, jnp.float32)
    tab_t = table.T
    tab_tail = lax.pad(lax.slice(tab_t, (0, TAIL1), (DIM, ROWS)),
                       jnp.float32(0), ((0, 0, 0), (0, 64, 0)))

    mesh = plsc.VectorSubcoreMesh(core_axis_name="c", subcore_axis_name="s")
    params = pltpu.CompilerParams(
        needs_layout_passes=False, use_tc_tiling_on_sc=True)

    route = pl.kernel(
        _route_kernel,
        mesh=mesh,
        compiler_params=params,
        out_type=jax.ShapeDtypeStruct((NENT, ZPAD), jnp.float32),
        scratch_types=[
            pltpu.VMEM((NENT,), jnp.int32),
            pltpu.VMEM((DIM, C), jnp.float32),
            pltpu.VMEM((DIM, C), jnp.float32),
            pltpu.VMEM((2 * WL_CAP,), jnp.int32),
            pltpu.VMEM((2 * WL_CAP,), jnp.int32),
            pltpu.VMEM((RL_CAP,), jnp.int32),
            pltpu.VMEM((RL_CAP,), jnp.int32),
            pltpu.VMEM((ZPAD, ZPAD), jnp.float32),
            pltpu.VMEM((ZPAD,), jnp.int32),
            pltpu.SemaphoreType.DMA,
            pltpu.SemaphoreType.DMA,
        ],
    )
    zbuf = route(ni, nj, tab_t, tab_tail)

    G = 2048
    return pl.pallas_call(
        _norm_tc_kernel,
        grid=(BATCH // G,),
        in_specs=[
            pl.BlockSpec((G, ZPAD), lambda i: (i, 0)),
            pl.BlockSpec((G, ZPAD), lambda i: (i + BATCH // G, 0)),
            pl.BlockSpec(memory_space=pltpu.SMEM),
        ],
        out_specs=pl.BlockSpec((G,), lambda i: (i,)),
        out_shape=jax.ShapeDtypeStruct((BATCH,), jnp.float32),
    )(zbuf, zbuf, jnp.full((1,), beta, jnp.float32))
